# Initial kernel scaffold; baseline (speedup 1.0000x reference)
import jax
import jax.numpy as jnp
from jax.experimental import pallas as pl


def kernel(x, stem_w, stem_b, l1b0_conv1_w, l1b0_conv1_b, l1b0_conv2_w, l1b0_conv2_b, l1b1_conv1_w, l1b1_conv1_b, l1b1_conv2_w, l1b1_conv2_b, l2b0_conv1_w, l2b0_conv1_b, l2b0_conv2_w, l2b0_conv2_b, l2b0_down_w, l2b0_down_b, l2b1_conv1_w, l2b1_conv1_b, l2b1_conv2_w, l2b1_conv2_b, l3b0_conv1_w, l3b0_conv1_b, l3b0_conv2_w, l3b0_conv2_b, l3b0_down_w, l3b0_down_b, l3b1_conv1_w, l3b1_conv1_b, l3b1_conv2_w, l3b1_conv2_b, l4b0_conv1_w, l4b0_conv1_b, l4b0_conv2_w, l4b0_conv2_b, l4b0_down_w, l4b0_down_b, l4b1_conv1_w, l4b1_conv1_b, l4b1_conv2_w, l4b1_conv2_b):
    raise NotImplementedError("write your pallas kernel here")


# speedup vs baseline: 22.2565x; 22.2565x over previous
import functools

import jax
import jax.numpy as jnp
from jax.experimental import pallas as pl
from jax.experimental.pallas import tpu as pltpu


def _s2(v, axis, start, n):
    sl = [slice(None)] * v.ndim
    sl[axis] = slice(start, start + 2 * n)
    v = v[tuple(sl)]
    shp = list(v.shape)
    shp[axis:axis + 1] = [n, 2]
    v = v.reshape(shp)
    idx = [slice(None)] * v.ndim
    idx[axis + 1] = 0
    return v[tuple(idx)]


def _conv_taps(xp, w_ref, oh, ow, stride, kh, kw, cin):
    bn = xp.shape[0]
    acc = None
    for ki in range(kh):
        if stride == 1:
            rows = xp[:, ki:ki + oh]
        else:
            rows = _s2(xp, 1, ki, oh)
        taps = []
        for kj in range(kw):
            if stride == 1:
                taps.append(rows[:, :, kj:kj + ow, :])
            else:
                taps.append(_s2(rows, 2, kj, ow))
        a = jnp.concatenate(taps, axis=-1).reshape(bn * oh * ow, kw * cin)
        t = jnp.dot(a, w_ref[ki], preferred_element_type=jnp.float32)
        acc = t if acc is None else acc + t
    return acc


def _block_kernel(*refs, stride, has_down, gap):
    if has_down:
        (x_ref, w1_ref, b1_ref, w2_ref, b2_ref, wd_ref, bd_ref,
         o_ref, xp_ref, mid_ref) = refs
    else:
        x_ref, w1_ref, b1_ref, w2_ref, b2_ref, o_ref, xp_ref, mid_ref = refs
    bn, H, W, C = x_ref.shape
    cout = b1_ref.shape[-1]
    oh, ow = H // stride, W // stride

    x = x_ref[...]
    xp_ref[...] = jnp.zeros_like(xp_ref)
    xp_ref[:, 1:H + 1, 1:W + 1, :] = x.astype(jnp.bfloat16)
    y1 = _conv_taps(xp_ref[...], w1_ref, oh, ow, stride, 3, 3, C)
    y1 = jnp.maximum(y1 + b1_ref[...], 0.0)

    mid_ref[...] = jnp.zeros_like(mid_ref)
    mid_ref[:, 1:oh + 1, 1:ow + 1, :] = (
        y1.reshape(bn, oh, ow, cout).astype(jnp.bfloat16))
    y2 = _conv_taps(mid_ref[...], w2_ref, oh, ow, 1, 3, 3, cout)

    if has_down:
        xc = _s2(_s2(x, 1, 0, oh), 2, 0, ow) if stride == 2 else x
        idv = jnp.dot(xc.astype(jnp.bfloat16).reshape(bn * oh * ow, C),
                      wd_ref[...], preferred_element_type=jnp.float32)
        idv = idv + bd_ref[...]
    else:
        idv = x.reshape(bn * oh * ow, C)

    out = jnp.maximum(y2 + b2_ref[...] + idv, 0.0)
    if gap:
        o_ref[...] = jnp.mean(out.reshape(bn, oh * ow, cout), axis=1)
    else:
        o_ref[...] = out.reshape(bn, oh, ow, cout)


def _run_block(x, w1, b1, w2, b2, wd, bd, *, stride, bn, gap=False):
    N, H, W, C = x.shape
    cout = b1.shape[-1]
    oh, ow = H // stride, W // stride
    w1r = w1[:9 * C].reshape(3, 3 * C, cout)
    w2r = w2[:9 * cout].reshape(3, 3 * cout, cout)
    has_down = wd is not None

    args = [x, w1r, b1, w2r, b2]
    in_specs = [
        pl.BlockSpec((bn, H, W, C), lambda i: (i, 0, 0, 0)),
        pl.BlockSpec((3, 3 * C, cout), lambda i: (0, 0, 0)),
        pl.BlockSpec((1, cout), lambda i: (0, 0)),
        pl.BlockSpec((3, 3 * cout, cout), lambda i: (0, 0, 0)),
        pl.BlockSpec((1, cout), lambda i: (0, 0)),
    ]
    if has_down:
        args += [wd, bd]
        in_specs += [pl.BlockSpec((C, cout), lambda i: (0, 0)),
                     pl.BlockSpec((1, cout), lambda i: (0, 0))]
    if gap:
        out_shape = jax.ShapeDtypeStruct((N, cout), jnp.float32)
        out_spec = pl.BlockSpec((bn, cout), lambda i: (i, 0))
    else:
        out_shape = jax.ShapeDtypeStruct((N, oh, ow, cout), jnp.float32)
        out_spec = pl.BlockSpec((bn, oh, ow, cout), lambda i: (i, 0, 0, 0))

    return pl.pallas_call(
        functools.partial(_block_kernel, stride=stride, has_down=has_down,
                          gap=gap),
        out_shape=out_shape,
        grid_spec=pltpu.PrefetchScalarGridSpec(
            num_scalar_prefetch=0,
            grid=(N // bn,),
            in_specs=in_specs,
            out_specs=out_spec,
            scratch_shapes=[
                pltpu.VMEM((bn, H + 2, W + 2, C), jnp.bfloat16),
                pltpu.VMEM((bn, oh + 2, ow + 2, cout), jnp.bfloat16),
            ]),
        compiler_params=pltpu.CompilerParams(
            dimension_semantics=("parallel",),
            vmem_limit_bytes=56 * 1024 * 1024),
    )(*args)


def _stem_kernel(x_ref, w_ref, b_ref, o_ref, yp_ref):
    bn = x_ref.shape[0]
    x = x_ref[...]
    acc = None
    for p in range(4):
        rows = x[:, p:p + 112]
        a = jnp.concatenate([rows[:, :, q:q + 112, :] for q in range(4)],
                            axis=-1).reshape(bn * 112 * 112, 48)
        t = jnp.dot(a, w_ref[p], preferred_element_type=jnp.float32)
        acc = t if acc is None else acc + t
    y = jnp.maximum(acc + b_ref[...], 0.0).reshape(bn, 112, 112, 64)
    yp_ref[...] = jnp.zeros_like(yp_ref)
    yp_ref[:, 1:113, 1:113, :] = y
    yp = yp_ref[...]
    out = None
    for u in range(3):
        for v in range(3):
            t = _s2(_s2(yp, 1, u, 56), 2, v, 56)
            out = t if out is None else jnp.maximum(out, t)
    o_ref[...] = out


def _stem(x_nchw, stem_w, stem_b, bn):
    N = x_nchw.shape[0]
    xp = jnp.pad(x_nchw, ((0, 0), (0, 0), (3, 5), (3, 5)))
    x2 = (xp.reshape(N, 3, 116, 2, 116, 2)
          .transpose(0, 2, 4, 3, 5, 1)
          .reshape(N, 116, 116, 12)
          .astype(jnp.bfloat16))
    w7 = stem_w[:147].reshape(7, 7, 3, 64)
    w8 = jnp.zeros((8, 8, 3, 64), stem_w.dtype).at[:7, :7].set(w7)
    w2r = (w8.reshape(4, 2, 4, 2, 3, 64)
           .transpose(0, 2, 1, 3, 4, 5)
           .reshape(4, 48, 64))

    return pl.pallas_call(
        _stem_kernel,
        out_shape=jax.ShapeDtypeStruct((N, 56, 56, 64), jnp.float32),
        grid_spec=pltpu.PrefetchScalarGridSpec(
            num_scalar_prefetch=0,
            grid=(N // bn,),
            in_specs=[
                pl.BlockSpec((bn, 116, 116, 12), lambda i: (i, 0, 0, 0)),
                pl.BlockSpec((4, 48, 64), lambda i: (0, 0, 0)),
                pl.BlockSpec((1, 64), lambda i: (0, 0)),
            ],
            out_specs=pl.BlockSpec((bn, 56, 56, 64), lambda i: (i, 0, 0, 0)),
            scratch_shapes=[pltpu.VMEM((bn, 114, 114, 64), jnp.float32)]),
        compiler_params=pltpu.CompilerParams(
            dimension_semantics=("parallel",),
            vmem_limit_bytes=56 * 1024 * 1024),
    )(x2, w2r, stem_b)


def kernel(x, stem_w, stem_b, l1b0_conv1_w, l1b0_conv1_b, l1b0_conv2_w, l1b0_conv2_b, l1b1_conv1_w, l1b1_conv1_b, l1b1_conv2_w, l1b1_conv2_b, l2b0_conv1_w, l2b0_conv1_b, l2b0_conv2_w, l2b0_conv2_b, l2b0_down_w, l2b0_down_b, l2b1_conv1_w, l2b1_conv1_b, l2b1_conv2_w, l2b1_conv2_b, l3b0_conv1_w, l3b0_conv1_b, l3b0_conv2_w, l3b0_conv2_b, l3b0_down_w, l3b0_down_b, l3b1_conv1_w, l3b1_conv1_b, l3b1_conv2_w, l3b1_conv2_b, l4b0_conv1_w, l4b0_conv1_b, l4b0_conv2_w, l4b0_conv2_b, l4b0_down_w, l4b0_down_b, l4b1_conv1_w, l4b1_conv1_b, l4b1_conv2_w, l4b1_conv2_b):
    h = _stem(x, stem_w, stem_b, bn=1)
    h = _run_block(h, l1b0_conv1_w, l1b0_conv1_b, l1b0_conv2_w, l1b0_conv2_b,
                   None, None, stride=1, bn=1)
    h = _run_block(h, l1b1_conv1_w, l1b1_conv1_b, l1b1_conv2_w, l1b1_conv2_b,
                   None, None, stride=1, bn=1)
    h = _run_block(h, l2b0_conv1_w, l2b0_conv1_b, l2b0_conv2_w, l2b0_conv2_b,
                   l2b0_down_w, l2b0_down_b, stride=2, bn=2)
    h = _run_block(h, l2b1_conv1_w, l2b1_conv1_b, l2b1_conv2_w, l2b1_conv2_b,
                   None, None, stride=1, bn=4)
    h = _run_block(h, l3b0_conv1_w, l3b0_conv1_b, l3b0_conv2_w, l3b0_conv2_b,
                   l3b0_down_w, l3b0_down_b, stride=2, bn=4)
    h = _run_block(h, l3b1_conv1_w, l3b1_conv1_b, l3b1_conv2_w, l3b1_conv2_b,
                   None, None, stride=1, bn=8)
    h = _run_block(h, l4b0_conv1_w, l4b0_conv1_b, l4b0_conv2_w, l4b0_conv2_b,
                   l4b0_down_w, l4b0_down_b, stride=2, bn=8)
    h = _run_block(h, l4b1_conv1_w, l4b1_conv1_b, l4b1_conv2_w, l4b1_conv2_b,
                   None, None, stride=1, bn=8, gap=True)
    return h
